# Initial kernel scaffold; baseline (speedup 1.0000x reference)
import jax
import jax.numpy as jnp
from jax.experimental import pallas as pl


def kernel(x, W, edge_index, counts, out_edge_index, layer_i):
    raise NotImplementedError("write your pallas kernel here")


# speedup vs baseline: 4.8499x; 4.8499x over previous
import functools

import jax
import jax.numpy as jnp
from jax import lax
from jax.experimental import pallas as pl
from jax.experimental.pallas import tpu as pltpu
from jax.experimental.pallas import tpu_sc as plsc

N_NODES = 10000
D = 128
N_EDGES = 320000

NC, NS = 2, 16
NW = NC * NS
CHUNK = 128
CHUNKS_PER_W = 79
EDGES_PER_W = CHUNK * CHUNKS_PER_W
E_PAD = NW * EDGES_PER_W
ACC_ROWS = 10240
DUMMY_ROW = N_NODES

ZERO_ROWS_PER_SUB = ACC_ROWS // NS
OUT_ROWS_PER_SUB = ACC_ROWS // NS


def _sc_aggregate(x, src, dst):
  mesh = plsc.VectorSubcoreMesh(core_axis_name="c", subcore_axis_name="s")

  @functools.partial(
      pl.kernel,
      out_type=jax.ShapeDtypeStruct((NC, ACC_ROWS, D), jnp.float32),
      mesh=mesh,
      scratch_types=[
          pltpu.VMEM((CHUNK,), jnp.int32),
          pltpu.VMEM((1, CHUNK), jnp.int32),
          pltpu.VMEM((CHUNK, D), jnp.float32),
          pltpu.VMEM_SHARED((ACC_ROWS, D), jnp.float32),
      ],
  )
  def agg(x_hbm, src_hbm, dst_hbm, out_hbm, s_idx, d_idx, rows, acc):
    cid = lax.axis_index("c")
    sid = lax.axis_index("s")
    wid = cid * NS + sid

    @pl.loop(0, CHUNK)
    def _(r):
      @pl.loop(0, D, step=16)
      def _(c):
        rows[r, pl.ds(c, 16)] = jnp.zeros((16,), jnp.float32)

    @pl.loop(0, ZERO_ROWS_PER_SUB // CHUNK)
    def _(k):
      pltpu.sync_copy(
          rows, acc.at[pl.ds(sid * ZERO_ROWS_PER_SUB + k * CHUNK, CHUNK)])

    plsc.subcore_barrier()

    base = wid * EDGES_PER_W

    @pl.loop(0, CHUNKS_PER_W)
    def _(j):
      off = base + j * CHUNK
      pltpu.sync_copy(src_hbm.at[pl.ds(off, CHUNK)], s_idx)
      pltpu.sync_copy(dst_hbm.at[pl.ds(off, CHUNK)], d_idx.at[0])
      pltpu.sync_copy(x_hbm.at[s_idx], rows)
      pltpu.sync_copy(rows, acc.at[d_idx.at[0]], add=True)

    plsc.subcore_barrier()

    rbase = sid * OUT_ROWS_PER_SUB
    pltpu.sync_copy(acc.at[pl.ds(rbase, OUT_ROWS_PER_SUB)],
                    out_hbm.at[cid, pl.ds(rbase, OUT_ROWS_PER_SUB)])

  return agg(x, src, dst)


def _tc_combine(partials, W):
  def body(p_ref, w_ref, o_ref):
    a = p_ref[0, :N_NODES] + p_ref[1, :N_NODES]
    o_ref[...] = lax.dot_general(
        a, w_ref[...], (((1,), (1,)), ((), ())),
        preferred_element_type=jnp.float32)

  return pl.pallas_call(
      body,
      out_shape=jax.ShapeDtypeStruct((N_NODES, D), jnp.float32),
  )(partials, W)


def kernel(x, W, edge_index, counts, out_edge_index, layer_i):
  del counts, out_edge_index, layer_i
  pad = E_PAD - N_EDGES
  src = jnp.concatenate([edge_index[0], jnp.zeros((pad,), jnp.int32)])
  dst = jnp.concatenate([edge_index[1],
                         jnp.full((pad,), DUMMY_ROW, jnp.int32)])
  partials = _sc_aggregate(x, src, dst)
  return _tc_combine(partials, W)
